# Initial kernel scaffold; baseline (speedup 1.0000x reference)
import jax
import jax.numpy as jnp
from jax.experimental import pallas as pl


def kernel(X_train, y, X_test, W_ctx, b_ctx, W_pred, b_pred):
    raise NotImplementedError("write your pallas kernel here")


# speedup vs baseline: 3.0423x; 3.0423x over previous
import functools

import jax
import jax.numpy as jnp
from jax import lax
from jax.experimental import pallas as pl
from jax.experimental.pallas import tpu as pltpu
from jax.experimental.pallas import tpu_sc as plsc

NC, NS = 2, 16
NW = NC * NS
CHUNK = 256
SCAT = 128


def _sc_segment_sum(X_train, y):
    N, D = X_train.shape
    assert N % CHUNK == 0 and D % 16 == 0 and CHUNK % SCAT == 0
    num_chunks = N // CHUNK
    iters = (num_chunks + NW - 1) // NW

    mesh = plsc.VectorSubcoreMesh(core_axis_name="c", subcore_axis_name="s")

    @functools.partial(
        pl.kernel,
        out_type=(
            jax.ShapeDtypeStruct((NW, 2, D), jnp.float32),
            jax.ShapeDtypeStruct((NW, 16), jnp.int32),
        ),
        mesh=mesh,
        scratch_types=[
            pltpu.VMEM((CHUNK, D), jnp.float32),
            pltpu.VMEM((CHUNK,), jnp.int32),
            pltpu.VMEM((CHUNK // SCAT, SCAT), jnp.int32),
            pltpu.VMEM((2, D), jnp.float32),
            pltpu.VMEM((16,), jnp.int32),
            pltpu.VMEM_SHARED((2 * NS, D), jnp.float32),
        ],
    )
    def seg_kernel(x_hbm, y_hbm, out_hbm, cnt_hbm,
                   xbuf, ybuf, idxbuf, zbuf, cntbuf, shared):
        c = lax.axis_index("c")
        s = lax.axis_index("s")
        wid = c * NS + s
        zeros16f = jnp.zeros((16,), jnp.float32)
        for r in range(2):
            for k in range(D // 16):
                zbuf[r, pl.ds(k * 16, 16)] = zeros16f
        cntbuf[...] = jnp.zeros((16,), jnp.int32)
        pltpu.sync_copy(zbuf, shared.at[pl.ds(2 * s, 2)])
        two_s = 2 * s

        def body(gi, carry):
            g = gi * NW + wid

            @pl.when(g < num_chunks)
            def _():
                base = g * CHUNK
                pltpu.sync_copy(x_hbm.at[pl.ds(base, CHUNK)], xbuf)
                pltpu.sync_copy(y_hbm.at[pl.ds(base, CHUNK)], ybuf)
                for k in range(CHUNK // 16):
                    yv = ybuf[pl.ds(k * 16, 16)]
                    cntbuf[...] = cntbuf[...] + yv
                    idxbuf[(k * 16) // SCAT,
                           pl.ds((k * 16) % SCAT, 16)] = yv + two_s
                for b in range(CHUNK // SCAT):
                    pltpu.sync_copy(xbuf.at[pl.ds(b * SCAT, SCAT)],
                                    shared.at[idxbuf.at[b]], add=True)
            return carry

        lax.fori_loop(0, iters, body, 0)
        pltpu.sync_copy(shared.at[pl.ds(2 * s, 2)], zbuf)
        pltpu.sync_copy(zbuf, out_hbm.at[wid])
        pltpu.sync_copy(cntbuf, cnt_hbm.at[wid])

    return seg_kernel(X_train, y)


def _mm_body(x_ref, w_ref, o_ref):
    o_ref[...] = jnp.dot(x_ref[...], w_ref[...],
                         preferred_element_type=jnp.float32)


def _partial_logits(X_test, W1):
    Nt, D = X_test.shape
    blk = 2000
    assert Nt % blk == 0
    return pl.pallas_call(
        _mm_body,
        grid=(Nt // blk,),
        in_specs=[pl.BlockSpec((blk, D), lambda i: (i, 0)),
                  pl.BlockSpec((D, 2), lambda i: (0, 0))],
        out_specs=pl.BlockSpec((blk, 2), lambda i: (i, 0)),
        out_shape=jax.ShapeDtypeStruct((Nt, 2), jnp.float32),
    )(X_test, W1)


def _combine_body(n_train, part_ref, p_ref, cnt_ref, wctx_ref, bctx_ref,
                  w2_ref, bpred_ref, o_ref):
    P = p_ref[...]
    Ssum = jnp.sum(P, axis=0, keepdims=True)
    D = Ssum.shape[1] // 2
    S0, S1 = Ssum[:, :D], Ssum[:, D:]
    c1 = jnp.sum(cnt_ref[...]).astype(jnp.float32)
    c0 = jnp.float32(n_train) - c1
    bctx = bctx_ref[...]
    Wc = wctx_ref[...]
    res0 = jnp.dot(S0, Wc, preferred_element_type=jnp.float32) + c0 * bctx
    res1 = jnp.dot(S1, Wc, preferred_element_type=jnp.float32) + c1 * bctx
    W2 = w2_ref[...]
    cvec = (jnp.dot(res0, W2[:D], preferred_element_type=jnp.float32)
            + jnp.dot(res1, W2[D:], preferred_element_type=jnp.float32)
            + bpred_ref[...])
    o_ref[...] = part_ref[...] + cvec


def _combine(partial, P, cnts2, W_ctx, bctx2, W2, bpred2, n_train):
    Nt = partial.shape[0]
    D = W_ctx.shape[0]
    blk = 8000
    assert Nt % blk == 0
    const = lambda i: (0, 0)
    return pl.pallas_call(
        functools.partial(_combine_body, n_train),
        grid=(Nt // blk,),
        in_specs=[pl.BlockSpec((blk, 2), lambda i: (i, 0)),
                  pl.BlockSpec(P.shape, const),
                  pl.BlockSpec(cnts2.shape, const),
                  pl.BlockSpec((D, D), const),
                  pl.BlockSpec((1, D), const),
                  pl.BlockSpec((2 * D, 2), const),
                  pl.BlockSpec((1, 2), const)],
        out_specs=pl.BlockSpec((blk, 2), lambda i: (i, 0)),
        out_shape=jax.ShapeDtypeStruct((Nt, 2), jnp.float32),
    )(partial, P, cnts2, W_ctx, bctx2, W2, bpred2)


def kernel(X_train, y, X_test, W_ctx, b_ctx, W_pred, b_pred):
    N, D = X_train.shape
    partials, cnts = _sc_segment_sum(X_train, y.astype(jnp.int32))
    P = partials.reshape(NW, 2 * D)
    cnts2 = cnts.reshape(4, 128)
    W1, W2 = W_pred[:D], W_pred[D:]
    partial = _partial_logits(X_test, W1)
    return _combine(partial, P, cnts2, W_ctx, b_ctx.reshape(1, D), W2,
                    b_pred.reshape(1, 2), N)
